# baseline (speedup 1.0000x reference)
import functools

import jax
import jax.numpy as jnp
from jax import lax
from jax.experimental import pallas as pl
from jax.experimental.pallas import tpu as pltpu
from jax.experimental.pallas import tpu_sc as plsc

NUM_CORES = 2
NUM_SUBCORES = 16
NUM_WORKERS = NUM_CORES * NUM_SUBCORES

CHUNK = 128
OUT_W = 128


def kernel(token_ids, weight):
    B, S = token_ids.shape
    V, D = weight.shape
    total = B * S
    per_worker = total // NUM_WORKERS
    n_chunks = per_worker // CHUNK

    idx = token_ids.reshape(NUM_WORKERS, n_chunks, CHUNK).astype(jnp.int32)

    mesh = plsc.VectorSubcoreMesh(core_axis_name="c", subcore_axis_name="s")

    @functools.partial(
        pl.kernel,
        mesh=mesh,
        out_type=jax.ShapeDtypeStruct((total, OUT_W), jnp.float32),
        scratch_types=[
            pltpu.VMEM((n_chunks, CHUNK), jnp.int32),
            pltpu.VMEM((CHUNK, D), jnp.float32),
            pltpu.SemaphoreType.DMA,
        ],
        compiler_params=pltpu.CompilerParams(use_tc_tiling_on_sc=False),
    )
    def gather_kernel(idx_hbm, table_hbm, out_hbm, idx_v, rows_v, sem):
        wid = lax.axis_index("s") * NUM_CORES + lax.axis_index("c")
        base = wid * per_worker
        pltpu.sync_copy(idx_hbm.at[wid], idx_v)

        def body(g, carry):
            pltpu.async_copy(table_hbm.at[idx_v.at[g]], rows_v, sem).wait()
            pltpu.sync_copy(
                rows_v, out_hbm.at[pl.ds(base + g * CHUNK, CHUNK), pl.ds(0, D)]
            )
            return carry

        lax.fori_loop(0, n_chunks, body, 0)

    out = gather_kernel(idx, weight)
    return out[:, :D].reshape(B, S, D)


# speedup vs baseline: 1.1482x; 1.1482x over previous
import functools

import jax
import jax.numpy as jnp
from jax import lax
from jax.experimental import pallas as pl
from jax.experimental.pallas import tpu as pltpu
from jax.experimental.pallas import tpu_sc as plsc

NUM_CORES = 2
NUM_SUBCORES = 16
NUM_WORKERS = NUM_CORES * NUM_SUBCORES

CHUNK = 128


def kernel(token_ids, weight):
    B, S = token_ids.shape
    V, D = weight.shape
    total = B * S
    per_worker = total // NUM_WORKERS
    n_chunks = per_worker // CHUNK
    n_pairs = n_chunks // 2

    idx = token_ids.reshape(NUM_WORKERS, n_chunks, CHUNK).astype(jnp.int32)

    mesh = plsc.VectorSubcoreMesh(core_axis_name="c", subcore_axis_name="s")

    @functools.partial(
        pl.kernel,
        mesh=mesh,
        out_type=jax.ShapeDtypeStruct((total, D), jnp.float32),
        scratch_types=[
            pltpu.VMEM((n_chunks, CHUNK), jnp.int32),
            pltpu.VMEM((CHUNK, D), jnp.float32),
            pltpu.VMEM((CHUNK, D), jnp.float32),
            pltpu.SemaphoreType.DMA,
            pltpu.SemaphoreType.DMA,
        ],
        compiler_params=pltpu.CompilerParams(use_tc_tiling_on_sc=False),
    )
    def gather_kernel(idx_hbm, table_hbm, out_hbm, idx_v, rows_a, rows_b, sem_a, sem_b):
        wid = lax.axis_index("s") * NUM_CORES + lax.axis_index("c")
        base = wid * per_worker
        pltpu.sync_copy(idx_hbm.at[wid], idx_v)

        pltpu.async_copy(table_hbm.at[idx_v.at[0]], rows_a, sem_a)

        def body(i, carry):
            g = 2 * i
            pltpu.async_copy(table_hbm.at[idx_v.at[g + 1]], rows_b, sem_b)
            pltpu.make_async_copy(table_hbm.at[idx_v.at[g]], rows_a, sem_a).wait()
            pltpu.sync_copy(rows_a, out_hbm.at[pl.ds(base + g * CHUNK, CHUNK)])

            @pl.when(i < n_pairs - 1)
            def _():
                pltpu.async_copy(table_hbm.at[idx_v.at[g + 2]], rows_a, sem_a)

            pltpu.make_async_copy(table_hbm.at[idx_v.at[g + 1]], rows_b, sem_b).wait()
            pltpu.sync_copy(rows_b, out_hbm.at[pl.ds(base + (g + 1) * CHUNK, CHUNK)])
            return carry

        lax.fori_loop(0, n_pairs, body, 0)

    out = gather_kernel(idx, weight)
    return out.reshape(B, S, D)
